# Initial kernel scaffold; baseline (speedup 1.0000x reference)
import jax
import jax.numpy as jnp
from jax.experimental import pallas as pl


def kernel(input_ids, word_em):
    raise NotImplementedError("write your pallas kernel here")


# speedup vs baseline: 7.5593x; 7.5593x over previous
import functools
import math

import jax
import jax.numpy as jnp
from jax import lax
from jax.experimental import pallas as pl
from jax.experimental.pallas import tpu as pltpu
from jax.experimental.pallas import tpu_sc as plsc

D = 128
B = 4096
L = 200
BL = B * L

NC = 2
NS = 16
NW = NC * NS
PER_W = BL // NW
CHUNK = 128
NCHUNK = PER_W // CHUNK
NBUF = 4

_SCALE = math.sqrt(float(D))
_TC_BLK = 1000


def _scale_body(x_ref, o_ref):
    o_ref[...] = x_ref[...] * _SCALE


def _scale_table(word_em):
    rows = word_em.shape[0]
    return pl.pallas_call(
        _scale_body,
        grid=(rows // _TC_BLK,),
        in_specs=[pl.BlockSpec((_TC_BLK, D), lambda i: (i, 0))],
        out_specs=pl.BlockSpec((_TC_BLK, D), lambda i: (i, 0)),
        out_shape=jax.ShapeDtypeStruct((rows, D), jnp.float32),
    )(word_em)


_MESH = plsc.VectorSubcoreMesh(core_axis_name="c", subcore_axis_name="s")


@functools.partial(
    pl.kernel,
    mesh=_MESH,
    out_type=jax.ShapeDtypeStruct((BL, D), jnp.float32),
    scratch_types=[
        pltpu.VMEM((NCHUNK, CHUNK), jnp.int32),
        pltpu.VMEM((NBUF, CHUNK, D), jnp.float32),
    ]
    + [pltpu.SemaphoreType.DMA] * (2 * NBUF),
)
def _gather_kernel(idx_hbm, table_hbm, out_hbm, idx_v, rows_v, *sems):
    gsems = sems[:NBUF]
    osems = sems[NBUF:]
    wid = lax.axis_index("s") * NC + lax.axis_index("c")
    base = wid * PER_W

    pltpu.sync_copy(idx_hbm.at[pl.ds(wid * NCHUNK, NCHUNK)], idx_v)

    def gather_start(j, b):
        pltpu.async_copy(table_hbm.at[idx_v.at[j]], rows_v.at[b], gsems[b])

    def gather_wait(b):
        pltpu.make_async_copy(
            table_hbm.at[pl.ds(0, CHUNK)], rows_v.at[b], gsems[b]
        ).wait()

    def out_start(j, b):
        pltpu.async_copy(
            rows_v.at[b], out_hbm.at[pl.ds(base + j * CHUNK, CHUNK)], osems[b]
        )

    def out_wait(b):
        pltpu.make_async_copy(
            out_hbm.at[pl.ds(0, CHUNK)], rows_v.at[b], osems[b]
        ).wait()

    for b in range(NBUF):
        gather_start(b, b)

    nsteps = NCHUNK // NBUF

    def body(s, carry):
        for b in range(NBUF):
            j = s * NBUF + b
            gather_wait(b)
            out_start(j, b)

            @pl.when(s < nsteps - 1)
            def _():
                out_wait(b)
                gather_start(j + NBUF, b)

        return carry

    lax.fori_loop(0, nsteps, body, 0)

    for b in range(NBUF):
        out_wait(b)


def kernel(input_ids, word_em):
    idx = input_ids.reshape(BL).astype(jnp.int32).reshape(BL // CHUNK, CHUNK)
    table = _scale_table(word_em)
    out = _gather_kernel(idx, table)
    return out.reshape(B, L, D)


# speedup vs baseline: 9.1178x; 1.2062x over previous
import functools
import math

import jax
import jax.numpy as jnp
from jax import lax
from jax.experimental import pallas as pl
from jax.experimental.pallas import tpu as pltpu
from jax.experimental.pallas import tpu_sc as plsc

D = 128
B = 4096
L = 200
BL = B * L

NC = 2
NS = 16
NW = NC * NS
PER_W = BL // NW
CHUNK = 128
NCHUNK = PER_W // CHUNK
NBUF = 4

_SCALE = math.sqrt(float(D))

_MESH = plsc.VectorSubcoreMesh(core_axis_name="c", subcore_axis_name="s")


@functools.partial(
    pl.kernel,
    mesh=_MESH,
    out_type=jax.ShapeDtypeStruct((BL, D), jnp.float32),
    scratch_types=[
        pltpu.VMEM((NCHUNK, CHUNK), jnp.int32),
        pltpu.VMEM((NBUF, CHUNK, D), jnp.float32),
    ]
    + [pltpu.SemaphoreType.DMA] * (2 * NBUF),
)
def _gather_kernel(idx_hbm, table_hbm, out_hbm, idx_v, rows_v, *sems):
    gsems = sems[:NBUF]
    osems = sems[NBUF:]
    wid = lax.axis_index("s") * NC + lax.axis_index("c")
    base = wid * PER_W

    pltpu.sync_copy(idx_hbm.at[pl.ds(wid * NCHUNK, NCHUNK)], idx_v)

    def gather_start(j, b):
        pltpu.async_copy(table_hbm.at[idx_v.at[j]], rows_v.at[b], gsems[b])

    def gather_wait(b):
        pltpu.make_async_copy(
            table_hbm.at[pl.ds(0, CHUNK)], rows_v.at[b], gsems[b]
        ).wait()

    def out_start(j, b):
        pltpu.async_copy(
            rows_v.at[b], out_hbm.at[pl.ds(base + j * CHUNK, CHUNK)], osems[b]
        )

    def out_wait(b):
        pltpu.make_async_copy(
            out_hbm.at[pl.ds(0, CHUNK)], rows_v.at[b], osems[b]
        ).wait()

    def scale_buf(b):
        def row_body(r, carry):
            for k in range(D // 16):
                sl = pl.ds(k * 16, 16)
                rows_v[b, r, sl] = rows_v[b, r, sl] * _SCALE
            return carry

        lax.fori_loop(0, CHUNK, row_body, 0)

    for b in range(NBUF):
        gather_start(b, b)

    nsteps = NCHUNK // NBUF

    def body(s, carry):
        for b in range(NBUF):
            j = s * NBUF + b
            gather_wait(b)
            scale_buf(b)
            out_start(j, b)

            @pl.when(s < nsteps - 1)
            def _():
                out_wait(b)
                gather_start(j + NBUF, b)

        return carry

    lax.fori_loop(0, nsteps, body, 0)

    for b in range(NBUF):
        out_wait(b)


def kernel(input_ids, word_em):
    idx = input_ids.reshape(BL).astype(jnp.int32).reshape(BL // CHUNK, CHUNK)
    out = _gather_kernel(idx, word_em)
    return out.reshape(B, L, D)


# speedup vs baseline: 9.1214x; 1.0004x over previous
import functools
import math

import jax
import jax.numpy as jnp
from jax import lax
from jax.experimental import pallas as pl
from jax.experimental.pallas import tpu as pltpu
from jax.experimental.pallas import tpu_sc as plsc

D = 128
B = 4096
L = 200
BL = B * L

NC = 2
NS = 16
NW = NC * NS
PER_W = BL // NW
CHUNK = 128
NCHUNK = PER_W // CHUNK
NBUF = 5

_SCALE = math.sqrt(float(D))

_MESH = plsc.VectorSubcoreMesh(core_axis_name="c", subcore_axis_name="s")


@functools.partial(
    pl.kernel,
    mesh=_MESH,
    out_type=jax.ShapeDtypeStruct((BL, D), jnp.float32),
    scratch_types=[
        pltpu.VMEM((NCHUNK, CHUNK), jnp.int32),
        pltpu.VMEM((NBUF, CHUNK, D), jnp.float32),
    ]
    + [pltpu.SemaphoreType.DMA] * (2 * NBUF),
)
def _gather_kernel(idx_hbm, table_hbm, out_hbm, idx_v, rows_v, *sems):
    gsems = sems[:NBUF]
    osems = sems[NBUF:]
    wid = lax.axis_index("s") * NC + lax.axis_index("c")
    base = wid * PER_W

    pltpu.sync_copy(idx_hbm.at[pl.ds(wid * NCHUNK, NCHUNK)], idx_v)

    def gather_start(j, b):
        pltpu.async_copy(table_hbm.at[idx_v.at[j]], rows_v.at[b], gsems[b])

    def gather_wait(b):
        pltpu.make_async_copy(
            table_hbm.at[pl.ds(0, CHUNK)], rows_v.at[b], gsems[b]
        ).wait()

    def out_start(j, b):
        pltpu.async_copy(
            rows_v.at[b], out_hbm.at[pl.ds(base + j * CHUNK, CHUNK)], osems[b]
        )

    def out_wait(b):
        pltpu.make_async_copy(
            out_hbm.at[pl.ds(0, CHUNK)], rows_v.at[b], osems[b]
        ).wait()

    def scale_buf(b):
        def row_body(r, carry):
            for k in range(D // 16):
                sl = pl.ds(k * 16, 16)
                rows_v[b, r, sl] = rows_v[b, r, sl] * _SCALE
            return carry

        lax.fori_loop(0, CHUNK, row_body, 0)

    for b in range(NBUF):
        gather_start(b, b)

    nsteps = NCHUNK // NBUF

    def body(s, carry):
        for b in range(NBUF):
            j = s * NBUF + b
            gather_wait(b)
            scale_buf(b)
            out_start(j, b)

            @pl.when(s < nsteps - 1)
            def _():
                out_wait(b)
                gather_start(j + NBUF, b)

        return carry

    lax.fori_loop(0, nsteps, body, 0)

    for b in range(NBUF):
        out_wait(b)


def kernel(input_ids, word_em):
    idx = input_ids.reshape(BL).astype(jnp.int32).reshape(BL // CHUNK, CHUNK)
    out = _gather_kernel(idx, word_em)
    return out.reshape(B, L, D)
